# Initial kernel scaffold; baseline (speedup 1.0000x reference)
import functools

import jax
import jax.numpy as jnp
from jax import lax
from jax.experimental import pallas as pl
from jax.experimental.pallas import tpu as pltpu
from jax.experimental.pallas import tpu_sc as plsc

N = 10000
E = 320000
D = 128
D_AUG = 144
NPAD = 10240
NW = 32
K = 128
EPW = 10112
NCHUNK = EPW // K
ROWS_PER_TILE = NPAD // 16


def _sc_segment_sum(x_aug, src, dst):
  mesh = plsc.VectorSubcoreMesh(core_axis_name="c", subcore_axis_name="s")

  @functools.partial(
      pl.kernel,
      mesh=mesh,
      out_type=jax.ShapeDtypeStruct((2 * NPAD, D_AUG), jnp.float32),
      scratch_types=[
          pltpu.VMEM((K,), jnp.int32),
          pltpu.VMEM((K,), jnp.int32),
          pltpu.VMEM((K, D_AUG), jnp.float32),
          pltpu.VMEM((K, D_AUG), jnp.float32),
          pltpu.VMEM_SHARED((NPAD, D_AUG), jnp.float32),
          pltpu.SemaphoreType.DMA,
      ],
  )
  def k(xa_hbm, src_hbm, dst_hbm, out_hbm,
        src_v, dst_v, rows_v, zeros_v, acc_sh, sem):
    cid = lax.axis_index("c")
    sid = lax.axis_index("s")
    wid = sid * 2 + cid

    def zrow(r, carry):
      for c in range(D_AUG // 16):
        zeros_v[r, pl.ds(c * 16, 16)] = jnp.zeros((16,), jnp.float32)
      return carry

    lax.fori_loop(0, K, zrow, 0)

    def zslab(j, carry):
      pltpu.sync_copy(zeros_v, acc_sh.at[pl.ds(sid * ROWS_PER_TILE + j * K, K)])
      return carry

    lax.fori_loop(0, ROWS_PER_TILE // K, zslab, 0)
    plsc.subcore_barrier()

    base0 = wid * EPW

    def step(i, carry):
      base = base0 + i * K
      pltpu.sync_copy(src_hbm.at[pl.ds(base, K)], src_v)
      pltpu.sync_copy(dst_hbm.at[pl.ds(base, K)], dst_v)
      pltpu.async_copy(xa_hbm.at[src_v], rows_v, sem).wait()
      pltpu.sync_copy(rows_v, acc_sh.at[dst_v], add=True)
      return carry

    lax.fori_loop(0, NCHUNK, step, 0)
    plsc.subcore_barrier()

    pltpu.sync_copy(
        acc_sh.at[pl.ds(sid * ROWS_PER_TILE, ROWS_PER_TILE)],
        out_hbm.at[pl.ds(cid * NPAD + sid * ROWS_PER_TILE, ROWS_PER_TILE)])

  return k(x_aug, src, dst)


def _tc_finish(acc0, acc1, x, W_l, b_l, W_r):
  BN = 1000

  def body(a0_ref, a1_ref, x_ref, wl_ref, wr_ref, b_ref, o_ref):
    s = a0_ref[...] + a1_ref[...]
    feat = s[:, :D]
    cnt = jnp.maximum(s[:, D:D + 1], 1.0)
    mean = feat / cnt
    o_ref[...] = (
        lax.dot_general(mean, wl_ref[...], (((1,), (1,)), ((), ())),
                        preferred_element_type=jnp.float32)
        + lax.dot_general(x_ref[...], wr_ref[...], (((1,), (1,)), ((), ())),
                          preferred_element_type=jnp.float32)
        + b_ref[...])

  return pl.pallas_call(
      body,
      grid=(N // BN,),
      in_specs=[
          pl.BlockSpec((BN, D_AUG), lambda i: (i, 0)),
          pl.BlockSpec((BN, D_AUG), lambda i: (i, 0)),
          pl.BlockSpec((BN, D), lambda i: (i, 0)),
          pl.BlockSpec((D, D), lambda i: (0, 0)),
          pl.BlockSpec((D, D), lambda i: (0, 0)),
          pl.BlockSpec((1, D), lambda i: (0, 0)),
      ],
      out_specs=pl.BlockSpec((BN, D), lambda i: (i, 0)),
      out_shape=jax.ShapeDtypeStruct((N, D), jnp.float32),
  )(acc0, acc1, x, W_l, W_r, b_l.reshape(1, D))


def kernel(x, edge_index, edge_attr, W_l, b_l, W_r):
  src = edge_index[0].astype(jnp.int32)
  dst = edge_index[1].astype(jnp.int32)

  x_aug = jnp.zeros((NPAD, D_AUG), jnp.float32)
  x_aug = x_aug.at[:N, :D].set(x.astype(jnp.float32))
  x_aug = x_aug.at[:N, D].set(1.0)

  pad = EPW * NW - E
  src_p = jnp.concatenate([src, jnp.zeros((pad,), jnp.int32)])
  dst_p = jnp.concatenate([dst, jnp.full((pad,), NPAD - 1, jnp.int32)])

  acc = _sc_segment_sum(x_aug, src_p, dst_p)
  return _tc_finish(acc[:NPAD], acc[NPAD:], x.astype(jnp.float32),
                    W_l, b_l, W_r)


# speedup vs baseline: 5.5774x; 5.5774x over previous
import functools

import jax
import jax.numpy as jnp
from jax import lax
from jax.experimental import pallas as pl
from jax.experimental.pallas import tpu as pltpu
from jax.experimental.pallas import tpu_sc as plsc

N = 10000
E = 320000
D = 128
D_AUG = 144
NPAD = 10240
NW = 32
K = 128
EPW = 10112
NCHUNK = EPW // K
ROWS_PER_TILE = NPAD // 16


def _sc_segment_sum(x_aug, src, dst):
  mesh = plsc.VectorSubcoreMesh(core_axis_name="c", subcore_axis_name="s")

  @functools.partial(
      pl.kernel,
      mesh=mesh,
      out_type=jax.ShapeDtypeStruct((2 * NPAD, D_AUG), jnp.float32),
      scratch_types=[
          pltpu.VMEM((K,), jnp.int32),
          pltpu.VMEM((K,), jnp.int32),
          pltpu.VMEM((K, D_AUG), jnp.float32),
          pltpu.VMEM((K, D_AUG), jnp.float32),
          pltpu.VMEM_SHARED((NPAD, D_AUG), jnp.float32),
          pltpu.SemaphoreType.DMA,
      ],
      compiler_params=pltpu.CompilerParams(use_tc_tiling_on_sc=False),
  )
  def k(xa_hbm, src_hbm, dst_hbm, out_hbm,
        src_v, dst_v, rows_v, zeros_v, acc_sh, sem):
    cid = lax.axis_index("c")
    sid = lax.axis_index("s")
    wid = sid * 2 + cid

    def zrow(r, carry):
      for c in range(D_AUG // 16):
        zeros_v[r, pl.ds(c * 16, 16)] = jnp.zeros((16,), jnp.float32)
      return carry

    lax.fori_loop(jnp.int32(0), jnp.int32(K), zrow, jnp.int32(0))

    def zslab(j, carry):
      pltpu.sync_copy(zeros_v, acc_sh.at[pl.ds(sid * ROWS_PER_TILE + j * K, K)])
      return carry

    lax.fori_loop(jnp.int32(0), jnp.int32(ROWS_PER_TILE // K), zslab, jnp.int32(0))
    plsc.subcore_barrier()

    base0 = wid * EPW

    def step(i, carry):
      base = base0 + i * K
      pltpu.sync_copy(src_hbm.at[pl.ds(base, K)], src_v)
      pltpu.sync_copy(dst_hbm.at[pl.ds(base, K)], dst_v)
      pltpu.async_copy(xa_hbm.at[src_v], rows_v, sem).wait()
      pltpu.sync_copy(rows_v, acc_sh.at[dst_v], add=True)
      return carry

    lax.fori_loop(jnp.int32(0), jnp.int32(NCHUNK), step, jnp.int32(0))
    plsc.subcore_barrier()

    pltpu.sync_copy(
        acc_sh.at[pl.ds(sid * ROWS_PER_TILE, ROWS_PER_TILE)],
        out_hbm.at[pl.ds(cid * NPAD + sid * ROWS_PER_TILE, ROWS_PER_TILE)])

  return k(x_aug, src, dst)


def _tc_finish(acc0, acc1, x, W_l, b_l, W_r):
  BN = 1000

  def body(a0_ref, a1_ref, x_ref, wl_ref, wr_ref, b_ref, o_ref):
    s = a0_ref[...] + a1_ref[...]
    feat = s[:, :D]
    cnt = jnp.maximum(s[:, D:D + 1], 1.0)
    mean = feat / cnt
    o_ref[...] = (
        lax.dot_general(mean, wl_ref[...], (((1,), (1,)), ((), ())),
                        preferred_element_type=jnp.float32)
        + lax.dot_general(x_ref[...], wr_ref[...], (((1,), (1,)), ((), ())),
                          preferred_element_type=jnp.float32)
        + b_ref[...])

  return pl.pallas_call(
      body,
      grid=(N // BN,),
      in_specs=[
          pl.BlockSpec((BN, D_AUG), lambda i: (i, jnp.int32(0))),
          pl.BlockSpec((BN, D_AUG), lambda i: (i, jnp.int32(0))),
          pl.BlockSpec((BN, D), lambda i: (i, jnp.int32(0))),
          pl.BlockSpec((D, D), lambda i: (jnp.int32(0), jnp.int32(0))),
          pl.BlockSpec((D, D), lambda i: (jnp.int32(0), jnp.int32(0))),
          pl.BlockSpec((1, D), lambda i: (jnp.int32(0), jnp.int32(0))),
      ],
      out_specs=pl.BlockSpec((BN, D), lambda i: (i, jnp.int32(0))),
      out_shape=jax.ShapeDtypeStruct((N, D), jnp.float32),
  )(acc0, acc1, x, W_l, W_r, b_l.reshape(1, D))


def kernel(x, edge_index, edge_attr, W_l, b_l, W_r):
  src = edge_index[0].astype(jnp.int32)
  dst = edge_index[1].astype(jnp.int32)

  x_aug = jnp.zeros((NPAD, D_AUG), jnp.float32)
  x_aug = x_aug.at[:N, :D].set(x.astype(jnp.float32))
  x_aug = x_aug.at[:N, D].set(1.0)

  pad = EPW * NW - E
  src_p = jnp.concatenate([src, jnp.zeros((pad,), jnp.int32)])
  dst_p = jnp.concatenate([dst, jnp.full((pad,), NPAD - 1, jnp.int32)])

  acc = _sc_segment_sum(x_aug, src_p, dst_p)
  out = _tc_finish(acc[:NPAD], acc[NPAD:], x.astype(jnp.float32),
                   W_l.astype(jnp.float32), b_l.astype(jnp.float32),
                   W_r.astype(jnp.float32))
  out_dtype = jnp.result_type(x.dtype, W_l.dtype)
  return out.astype(out_dtype)
